# Initial kernel scaffold; baseline (speedup 1.0000x reference)
import jax
import jax.numpy as jnp
from jax.experimental import pallas as pl


def kernel(hidden_states, weight, bias, cls_index):
    raise NotImplementedError("write your pallas kernel here")


# speedup vs baseline: 145.5904x; 145.5904x over previous
import functools

import jax
import jax.numpy as jnp
from jax.experimental import pallas as pl
from jax.experimental.pallas import tpu as pltpu


def _gather_head_body(idx_ref, h_ref, w_ref, b_ref, o_ref, rows_ref, sem,
                      *, rows_per_prog, seq_len):
    base = pl.program_id(0) * rows_per_prog
    for i in range(rows_per_prog):
        e = base + i
        src = e * seq_len + idx_ref[e]
        pltpu.make_async_copy(
            h_ref.at[pl.ds(src, 1)], rows_ref.at[pl.ds(i, 1)], sem
        ).start()
    pltpu.make_async_copy(h_ref.at[pl.ds(0, rows_per_prog)], rows_ref, sem).wait()
    acc = jax.lax.dot_general(
        rows_ref[...], w_ref[...],
        dimension_numbers=(((1,), (1,)), ((), ())),
        preferred_element_type=jnp.float32,
    )
    o_ref[...] = (acc + b_ref[...]).astype(o_ref.dtype)


def kernel(hidden_states, weight, bias, cls_index):
    B, S, H = hidden_states.shape
    out_dtype = hidden_states.dtype
    num_progs = 2
    rows_per_prog = B // num_progs

    h_flat = hidden_states.reshape(B * S, H)
    idx = cls_index.astype(jnp.int32)
    b2d = bias.reshape(1, 2).astype(jnp.float32)

    grid_spec = pltpu.PrefetchScalarGridSpec(
        num_scalar_prefetch=1,
        grid=(num_progs,),
        in_specs=[
            pl.BlockSpec(memory_space=pl.ANY),
            pl.BlockSpec((2, H), lambda g, idx_ref: (0, 0)),
            pl.BlockSpec((1, 2), lambda g, idx_ref: (0, 0)),
        ],
        out_specs=pl.BlockSpec((rows_per_prog, 2), lambda g, idx_ref: (g, 0)),
        scratch_shapes=[
            pltpu.VMEM((rows_per_prog, H), hidden_states.dtype),
            pltpu.SemaphoreType.DMA,
        ],
    )
    out = pl.pallas_call(
        functools.partial(
            _gather_head_body, rows_per_prog=rows_per_prog, seq_len=S
        ),
        out_shape=jax.ShapeDtypeStruct((B, 2), out_dtype),
        grid_spec=grid_spec,
        compiler_params=pltpu.CompilerParams(
            dimension_semantics=("parallel",),
            disable_bounds_checks=True,
        ),
    )(idx, h_flat, weight, b2d)
    return out
